# baseline (speedup 1.0000x reference)
import functools

import jax
import jax.numpy as jnp
from jax import lax
from jax.experimental import pallas as pl
from jax.experimental.pallas import tpu as pltpu
from jax.experimental.pallas import tpu_sc as plsc

N = 10000
E = 320000
D = 128
DH = D // 2
G = 64
CLS = 32

NC = 2
NS = 16
NW = NC * NS
C = 100
NBUF = 4
EPW = E // NW
NCHD = EPW // C
EPT = E // NS
NCH = EPT // C
RPT = N // NS
ZROWS = 125
WB = 632
WBL = N - (NS - 1) * WB
DEGW = 16

TB = 2000
HI = lax.Precision.DEFAULT

_mesh = plsc.VectorSubcoreMesh(core_axis_name="c", subcore_axis_name="s")


def _writeback(acc, out_hbm, cid, sid):
    r0 = pl.multiple_of(sid * WB, 8)

    @pl.when(sid < NS - 1)
    def _():
        pltpu.sync_copy(acc.at[pl.ds(r0, WB)], out_hbm.at[cid, pl.ds(r0, WB)])

    @pl.when(sid == NS - 1)
    def _():
        pltpu.sync_copy(acc.at[pl.ds(r0, WBL)], out_hbm.at[cid, pl.ds(r0, WBL)])


@functools.partial(
    pl.kernel,
    out_type=jax.ShapeDtypeStruct((NC, N, DEGW), jnp.float32),
    mesh=_mesh,
    scratch_types=[
        pltpu.VMEM((NCHD, C), jnp.int32),
        pltpu.VMEM((C, DEGW), jnp.float32),
        pltpu.VMEM((ZROWS, DEGW), jnp.float32),
        pltpu.VMEM_SHARED((N, DEGW), jnp.float32),
        pltpu.SemaphoreType.DMA,
    ],
)
def _deg_kernel(dst_hbm, out_hbm, didx, ones_buf, zbuf, acc, sem):
    cid = lax.axis_index("c")
    sid = lax.axis_index("s")
    wid = cid * NS + sid

    one = jnp.ones((16,), jnp.float32)
    zero = jnp.zeros((16,), jnp.float32)

    idx_cp = pltpu.async_copy(dst_hbm.at[wid], didx, sem)

    def fill(i, _):
        ones_buf[i, :] = one
        zbuf[i, :] = zero
        return 0
    lax.fori_loop(0, C, fill, 0)

    r0 = sid * RPT
    for k in range(RPT // ZROWS):
        pltpu.sync_copy(zbuf, acc.at[pl.ds(r0 + k * ZROWS, ZROWS)])
    idx_cp.wait()
    plsc.subcore_barrier()

    def chunk(i, _):
        pltpu.sync_copy(ones_buf, acc.at[didx.at[i]], add=True)
        return 0
    lax.fori_loop(0, NCHD, chunk, 0)
    plsc.subcore_barrier()

    _writeback(acc, out_hbm, cid, sid)


@functools.partial(
    pl.kernel,
    out_type=jax.ShapeDtypeStruct((NC, N, DH), jnp.float32),
    mesh=_mesh,
    scratch_types=[
        pltpu.VMEM((NCH, C), jnp.int32),
        pltpu.VMEM((NCH, C), jnp.int32),
        [pltpu.VMEM((C, DH), jnp.float32) for _ in range(NBUF)],
        pltpu.VMEM((ZROWS, DH), jnp.float32),
        pltpu.VMEM_SHARED((N, DH), jnp.float32),
        [pltpu.SemaphoreType.DMA for _ in range(NBUF)],
        [pltpu.SemaphoreType.DMA for _ in range(NBUF)],
    ],
    compiler_params=pltpu.CompilerParams(use_tc_tiling_on_sc=False),
)
def _segsum_kernel(xs_hbm, src_hbm, dst_hbm, out_hbm,
                   sidx, didx, rows, zbuf, acc, sem_g, sem_s):
    cid = lax.axis_index("c")
    sid = lax.axis_index("s")
    wid = cid * NS + sid

    zero = jnp.zeros((16,), jnp.float32)

    cp_s = pltpu.async_copy(src_hbm.at[cid, sid], sidx, sem_g[0])
    cp_d = pltpu.async_copy(dst_hbm.at[sid], didx, sem_g[1])
    xs_core = xs_hbm

    def fill(i, _):
        for jj in range(DH // 16):
            zbuf[i, pl.ds(jj * 16, 16)] = zero
        return 0
    lax.fori_loop(0, ZROWS, fill, 0)

    r0 = sid * RPT
    for k in range(RPT // ZROWS):
        pltpu.sync_copy(zbuf, acc.at[pl.ds(r0 + k * ZROWS, ZROWS)])
    cp_s.wait()
    cp_d.wait()

    for k in range(NBUF):
        pltpu.async_copy(xs_core.at[sidx.at[k]], rows[k], sem_g[k])
    plsc.subcore_barrier()

    def body(j, _):
        i0 = NBUF * j
        for k in range(NBUF):
            pltpu.make_async_copy(xs_core.at[sidx.at[i0 + k]], rows[k],
                                  sem_g[k]).wait()
            pltpu.sync_copy(rows[k], acc.at[didx.at[i0 + k]], add=True)
            nxt = lax.rem(i0 + NBUF + k, NCH)
            pltpu.async_copy(xs_core.at[sidx.at[nxt]], rows[k], sem_g[k])
        return 0
    lax.fori_loop(0, NCH // NBUF, body, 0)
    for k in range(NBUF):
        pltpu.make_async_copy(xs_core.at[sidx.at[k]], rows[k], sem_g[k]).wait()
    plsc.subcore_barrier()

    _writeback(acc, out_hbm, cid, sid)


def _dinv_of(deg_ref):
    deg = deg_ref[0, :, 0:1] + deg_ref[1, :, 0:1] + 1.0
    return lax.rsqrt(deg)


def _split_store(o_ref, r):
    o_ref[0] = r[:, :DH]
    o_ref[1] = r[:, DH:]


def _mm_scale_body(deg_ref, x_ref, w_ref, o_ref):
    dinv = _dinv_of(deg_ref)
    _split_store(o_ref, jnp.dot(x_ref[...], w_ref[...],
                                preferred_element_type=jnp.float32,
                                precision=HI) * dinv)


def _mm_scale(x, W, deg2):
    return pl.pallas_call(
        _mm_scale_body,
        grid=(N // TB,),
        in_specs=[
            pl.BlockSpec((NC, TB, DEGW), lambda i: (0, i, 0)),
            pl.BlockSpec((TB, D), lambda i: (i, 0)),
            pl.BlockSpec((D, D), lambda i: (0, 0)),
        ],
        out_specs=pl.BlockSpec((NC, TB, DH), lambda i: (0, i, 0)),
        out_shape=jax.ShapeDtypeStruct((NC, N, DH), jnp.float32),
    )(deg2, x, W)


def _combine_body(deg_ref, acc_ref, xs_ref, b_ref, w_ref, o_ref):
    dinv = _dinv_of(deg_ref)
    s = jnp.concatenate([acc_ref[0] + xs_ref[0], acc_ref[1] + xs_ref[1]], axis=1)
    h = jnp.maximum(s * dinv + b_ref[...], 0.0)
    _split_store(o_ref, jnp.dot(h, w_ref[...],
                                preferred_element_type=jnp.float32,
                                precision=HI) * dinv)


def _combine(acc2, xs, deg2, b, Wn):
    return pl.pallas_call(
        _combine_body,
        grid=(N // TB,),
        in_specs=[
            pl.BlockSpec((NC, TB, DEGW), lambda i: (0, i, 0)),
            pl.BlockSpec((NC, TB, DH), lambda i: (0, i, 0)),
            pl.BlockSpec((NC, TB, DH), lambda i: (0, i, 0)),
            pl.BlockSpec((1, D), lambda i: (0, 0)),
            pl.BlockSpec((D, D), lambda i: (0, 0)),
        ],
        out_specs=pl.BlockSpec((NC, TB, DH), lambda i: (0, i, 0)),
        out_shape=jax.ShapeDtypeStruct((NC, N, DH), jnp.float32),
    )(deg2, acc2, xs, b, Wn)


def _final_body(deg_ref, acc_ref, xs_ref, b_ref, batch_ref,
                wl1_ref, bl1_ref, wl2_ref, bl2_ref, o_ref, sums, cnts):
    i = pl.program_id(0)

    @pl.when(i == 0)
    def _init():
        sums[...] = jnp.zeros_like(sums)
        cnts[...] = jnp.zeros_like(cnts)

    dinv = _dinv_of(deg_ref)
    s = jnp.concatenate([acc_ref[0] + xs_ref[0], acc_ref[1] + xs_ref[1]], axis=1)
    h = jnp.maximum(s * dinv + b_ref[...], 0.0)
    gid = lax.broadcasted_iota(jnp.int32, (G, 1), 0)
    P = (batch_ref[0] == gid).astype(jnp.float32)
    sums[...] += jnp.dot(P, h, preferred_element_type=jnp.float32, precision=HI)
    cnts[...] += jnp.broadcast_to(jnp.sum(P, axis=1, keepdims=True), (G, D))

    @pl.when(i == N // TB - 1)
    def _head():
        g = sums[...] / jnp.maximum(cnts[...], 1.0)
        g1 = jnp.maximum(jnp.dot(g, wl1_ref[...],
                                 preferred_element_type=jnp.float32,
                                 precision=HI) + bl1_ref[...], 0.0)
        o_ref[...] = jnp.dot(g1, wl2_ref[...],
                             preferred_element_type=jnp.float32,
                             precision=HI) + bl2_ref[...]


def _final(acc2, xs, deg2, b, batch2, Wl1, bl1, Wl2, bl2):
    return pl.pallas_call(
        _final_body,
        grid=(N // TB,),
        in_specs=[
            pl.BlockSpec((NC, TB, DEGW), lambda i: (0, i, 0)),
            pl.BlockSpec((NC, TB, DH), lambda i: (0, i, 0)),
            pl.BlockSpec((NC, TB, DH), lambda i: (0, i, 0)),
            pl.BlockSpec((1, D), lambda i: (0, 0)),
            pl.BlockSpec((1, 1, TB), lambda i: (i, 0, 0)),
            pl.BlockSpec((D, D), lambda i: (0, 0)),
            pl.BlockSpec((1, D), lambda i: (0, 0)),
            pl.BlockSpec((D, CLS), lambda i: (0, 0)),
            pl.BlockSpec((1, CLS), lambda i: (0, 0)),
        ],
        out_specs=pl.BlockSpec((G, CLS), lambda i: (0, 0)),
        out_shape=jax.ShapeDtypeStruct((G, CLS), jnp.float32),
        scratch_shapes=[
            pltpu.VMEM((G, D), jnp.float32),
            pltpu.VMEM((G, D), jnp.float32),
        ],
    )(deg2, acc2, xs, b, batch2, Wl1, bl1, Wl2, bl2)


def kernel(x, edge_index, batch, W1, b1, W2, b2, W3, b3, Wl1, bl1, Wl2, bl2):
    src = edge_index[0].astype(jnp.int32)
    dst = edge_index[1].astype(jnp.int32)
    dstd = dst.reshape(NW, NCHD, C)
    srcb = jnp.stack([src, src + N]).reshape(NC, NS, NCH, C)
    dstb = dst.reshape(NS, NCH, C)
    batch2 = batch.astype(jnp.int32).reshape(N // TB, 1, TB)
    b1r, b2r, b3r = b1.reshape(1, D), b2.reshape(1, D), b3.reshape(1, D)
    bl1r, bl2r = bl1.reshape(1, D), bl2.reshape(1, CLS)

    deg2 = _deg_kernel(dstd)

    def segsum(xs):
        return _segsum_kernel(xs.reshape(NC * N, DH), srcb, dstb)

    xs1 = _mm_scale(x, W1, deg2)
    acc1 = segsum(xs1)
    xs2 = _combine(acc1, xs1, deg2, b1r, W2)
    acc2 = segsum(xs2)
    xs3 = _combine(acc2, xs2, deg2, b2r, W3)
    acc3 = segsum(xs3)
    return _final(acc3, xs3, deg2, b3r, batch2, Wl1, bl1r, Wl2, bl2r)


# speedup vs baseline: 1.1343x; 1.1343x over previous
import functools

import jax
import jax.numpy as jnp
from jax import lax
from jax.experimental import pallas as pl
from jax.experimental.pallas import tpu as pltpu
from jax.experimental.pallas import tpu_sc as plsc

N = 10000
NH = N // 2
E = 320000
D = 128
DH = D // 2
G = 64
CLS = 32

NC = 2
NS = 16
NW = NC * NS
C = 100
NBUF = 4
EPW = E // NW
NCHD = EPW // C
EPT = E // NS
NCH = EPT // C
RPT = N // NS
ZROWS = 125
WB = 632
WBL = N - (NS - 1) * WB
DEGW = 64

TBH = 1000
NGS = NH // TBH
HI = lax.Precision.DEFAULT

_mesh = plsc.VectorSubcoreMesh(core_axis_name="c", subcore_axis_name="s")


def _writeback(acc, out_hbm, cid, sid):
    r0 = pl.multiple_of(sid * WB, 8)

    @pl.when(sid < NS - 1)
    def _():
        pltpu.sync_copy(acc.at[pl.ds(r0, WB)], out_hbm.at[cid, pl.ds(r0, WB)])

    @pl.when(sid == NS - 1)
    def _():
        pltpu.sync_copy(acc.at[pl.ds(r0, WBL)], out_hbm.at[cid, pl.ds(r0, WBL)])


@functools.partial(
    pl.kernel,
    out_type=jax.ShapeDtypeStruct((NC, N, DEGW), jnp.float32),
    mesh=_mesh,
    scratch_types=[
        pltpu.VMEM((NCHD, C), jnp.int32),
        pltpu.VMEM((C, DEGW), jnp.float32),
        pltpu.VMEM((ZROWS, DEGW), jnp.float32),
        pltpu.VMEM_SHARED((N, DEGW), jnp.float32),
        pltpu.SemaphoreType.DMA,
    ],
    compiler_params=pltpu.CompilerParams(use_tc_tiling_on_sc=False),
)
def _deg_kernel(dst_hbm, out_hbm, didx, ones_buf, zbuf, acc, sem):
    cid = lax.axis_index("c")
    sid = lax.axis_index("s")
    wid = cid * NS + sid

    one = jnp.ones((16,), jnp.float32)
    zero = jnp.zeros((16,), jnp.float32)

    idx_cp = pltpu.async_copy(dst_hbm.at[wid], didx, sem)

    def fill_z(i, _):
        for jj in range(DEGW // 16):
            zbuf[i, pl.ds(jj * 16, 16)] = zero
        return 0
    lax.fori_loop(0, ZROWS, fill_z, 0)

    def fill_o(i, _):
        for jj in range(DEGW // 16):
            ones_buf[i, pl.ds(jj * 16, 16)] = one
        return 0
    lax.fori_loop(0, C, fill_o, 0)

    r0 = sid * RPT
    for k in range(RPT // ZROWS):
        pltpu.sync_copy(zbuf, acc.at[pl.ds(r0 + k * ZROWS, ZROWS)])
    idx_cp.wait()
    plsc.subcore_barrier()

    def chunk(i, _):
        pltpu.sync_copy(ones_buf, acc.at[didx.at[i]], add=True)
        return 0
    lax.fori_loop(0, NCHD, chunk, 0)
    plsc.subcore_barrier()

    _writeback(acc, out_hbm, cid, sid)


@functools.partial(
    pl.kernel,
    out_type=jax.ShapeDtypeStruct((NC, N, DH), jnp.float32),
    mesh=_mesh,
    scratch_types=[
        pltpu.VMEM((NCH, C), jnp.int32),
        pltpu.VMEM((NCH, C), jnp.int32),
        [pltpu.VMEM((C, DH), jnp.float32) for _ in range(NBUF)],
        pltpu.VMEM((ZROWS, DH), jnp.float32),
        pltpu.VMEM_SHARED((N, DH), jnp.float32),
        [pltpu.SemaphoreType.DMA for _ in range(NBUF)],
        [pltpu.SemaphoreType.DMA for _ in range(NBUF)],
    ],
    compiler_params=pltpu.CompilerParams(use_tc_tiling_on_sc=False),
)
def _segsum_kernel(xs_hbm, src_hbm, dst_hbm, out_hbm,
                   sidx, didx, rows, zbuf, acc, sem_g, sem_s):
    cid = lax.axis_index("c")
    sid = lax.axis_index("s")

    zero = jnp.zeros((16,), jnp.float32)

    cp_s = pltpu.async_copy(src_hbm.at[cid, sid], sidx, sem_g[0])
    cp_d = pltpu.async_copy(dst_hbm.at[sid], didx, sem_g[1])

    def fill(i, _):
        for jj in range(DH // 16):
            zbuf[i, pl.ds(jj * 16, 16)] = zero
        return 0
    lax.fori_loop(0, ZROWS, fill, 0)

    r0 = sid * RPT
    for k in range(RPT // ZROWS):
        pltpu.sync_copy(zbuf, acc.at[pl.ds(r0 + k * ZROWS, ZROWS)])
    cp_s.wait()
    cp_d.wait()

    for k in range(NBUF):
        pltpu.async_copy(xs_hbm.at[sidx.at[k]], rows[k], sem_g[k])
    plsc.subcore_barrier()

    def body(j, _):
        i0 = NBUF * j
        for k in range(NBUF):
            pltpu.make_async_copy(xs_hbm.at[sidx.at[i0 + k]], rows[k],
                                  sem_g[k]).wait()
            pltpu.sync_copy(rows[k], acc.at[didx.at[i0 + k]], add=True)
            nxt = lax.rem(i0 + NBUF + k, NCH)
            pltpu.async_copy(xs_hbm.at[sidx.at[nxt]], rows[k], sem_g[k])
        return 0
    lax.fori_loop(0, NCH // NBUF, body, 0)
    for k in range(NBUF):
        pltpu.make_async_copy(xs_hbm.at[sidx.at[k]], rows[k], sem_g[k]).wait()
    plsc.subcore_barrier()

    _writeback(acc, out_hbm, cid, sid)


def _dinvs(deg_ref):
    dA = deg_ref[0, :, 0:1] + deg_ref[1, :, 0:1] + 1.0
    dB = deg_ref[0, :, DH:DH + 1] + deg_ref[1, :, DH:DH + 1] + 1.0
    return lax.rsqrt(dA), lax.rsqrt(dB)


def _pack_store(o_ref, yA, yB):
    o_ref[0] = jnp.concatenate([yA[:, :DH], yB[:, :DH]], axis=1)
    o_ref[1] = jnp.concatenate([yA[:, DH:], yB[:, DH:]], axis=1)


def _unpack(p_ref):
    a = jnp.concatenate([p_ref[0, :, :DH], p_ref[1, :, :DH]], axis=1)
    b = jnp.concatenate([p_ref[0, :, DH:], p_ref[1, :, DH:]], axis=1)
    return a, b


def _mm_scale_body(deg_ref, xA_ref, xB_ref, w_ref, o_ref):
    dinvA, dinvB = _dinvs(deg_ref)
    yA = jnp.dot(xA_ref[...], w_ref[...],
                 preferred_element_type=jnp.float32, precision=HI) * dinvA
    yB = jnp.dot(xB_ref[...], w_ref[...],
                 preferred_element_type=jnp.float32, precision=HI) * dinvB
    _pack_store(o_ref, yA, yB)


def _mm_scale(x, W, degp):
    return pl.pallas_call(
        _mm_scale_body,
        grid=(NGS,),
        in_specs=[
            pl.BlockSpec((NC, TBH, D), lambda i: (0, i, 0)),
            pl.BlockSpec((TBH, D), lambda i: (i, 0)),
            pl.BlockSpec((TBH, D), lambda i: (NGS + i, 0)),
            pl.BlockSpec((D, D), lambda i: (0, 0)),
        ],
        out_specs=pl.BlockSpec((NC, TBH, D), lambda i: (0, i, 0)),
        out_shape=jax.ShapeDtypeStruct((NC, NH, D), jnp.float32),
    )(degp, x, x, W)


def _halves(deg_ref, acc_ref, xs_ref, b_ref):
    dinvA, dinvB = _dinvs(deg_ref)
    sA, sB = _unpack(acc_ref)
    xA, xB = _unpack(xs_ref)
    hA = jnp.maximum((sA + xA) * dinvA + b_ref[...], 0.0)
    hB = jnp.maximum((sB + xB) * dinvB + b_ref[...], 0.0)
    return hA, hB, dinvA, dinvB


def _combine_body(deg_ref, acc_ref, xs_ref, b_ref, w_ref, o_ref):
    hA, hB, dinvA, dinvB = _halves(deg_ref, acc_ref, xs_ref, b_ref)
    yA = jnp.dot(hA, w_ref[...],
                 preferred_element_type=jnp.float32, precision=HI) * dinvA
    yB = jnp.dot(hB, w_ref[...],
                 preferred_element_type=jnp.float32, precision=HI) * dinvB
    _pack_store(o_ref, yA, yB)


def _combine(accp, xsp, degp, b, Wn):
    return pl.pallas_call(
        _combine_body,
        grid=(NGS,),
        in_specs=[
            pl.BlockSpec((NC, TBH, D), lambda i: (0, i, 0)),
            pl.BlockSpec((NC, TBH, D), lambda i: (0, i, 0)),
            pl.BlockSpec((NC, TBH, D), lambda i: (0, i, 0)),
            pl.BlockSpec((1, D), lambda i: (0, 0)),
            pl.BlockSpec((D, D), lambda i: (0, 0)),
        ],
        out_specs=pl.BlockSpec((NC, TBH, D), lambda i: (0, i, 0)),
        out_shape=jax.ShapeDtypeStruct((NC, NH, D), jnp.float32),
    )(degp, accp, xsp, b, Wn)


def _final_body(deg_ref, acc_ref, xs_ref, b_ref, batchA_ref, batchB_ref,
                wl1_ref, bl1_ref, wl2_ref, bl2_ref, o_ref, sums, cnts):
    i = pl.program_id(0)

    @pl.when(i == 0)
    def _init():
        sums[...] = jnp.zeros_like(sums)
        cnts[...] = jnp.zeros_like(cnts)

    hA, hB, _, _ = _halves(deg_ref, acc_ref, xs_ref, b_ref)
    gid = lax.broadcasted_iota(jnp.int32, (G, 1), 0)
    PA = (batchA_ref[0] == gid).astype(jnp.float32)
    PB = (batchB_ref[0] == gid).astype(jnp.float32)
    sums[...] += (jnp.dot(PA, hA, preferred_element_type=jnp.float32,
                          precision=HI)
                  + jnp.dot(PB, hB, preferred_element_type=jnp.float32,
                            precision=HI))
    cnt = jnp.sum(PA, axis=1, keepdims=True) + jnp.sum(PB, axis=1,
                                                       keepdims=True)
    cnts[...] += jnp.broadcast_to(cnt, (G, D))

    @pl.when(i == NGS - 1)
    def _head():
        g = sums[...] / jnp.maximum(cnts[...], 1.0)
        g1 = jnp.maximum(jnp.dot(g, wl1_ref[...],
                                 preferred_element_type=jnp.float32,
                                 precision=HI) + bl1_ref[...], 0.0)
        o_ref[...] = jnp.dot(g1, wl2_ref[...],
                             preferred_element_type=jnp.float32,
                             precision=HI) + bl2_ref[...]


def _final(accp, xsp, degp, b, batch3, Wl1, bl1, Wl2, bl2):
    return pl.pallas_call(
        _final_body,
        grid=(NGS,),
        in_specs=[
            pl.BlockSpec((NC, TBH, D), lambda i: (0, i, 0)),
            pl.BlockSpec((NC, TBH, D), lambda i: (0, i, 0)),
            pl.BlockSpec((NC, TBH, D), lambda i: (0, i, 0)),
            pl.BlockSpec((1, D), lambda i: (0, 0)),
            pl.BlockSpec((1, 1, TBH), lambda i: (i, 0, 0)),
            pl.BlockSpec((1, 1, TBH), lambda i: (NGS + i, 0, 0)),
            pl.BlockSpec((D, D), lambda i: (0, 0)),
            pl.BlockSpec((1, D), lambda i: (0, 0)),
            pl.BlockSpec((D, CLS), lambda i: (0, 0)),
            pl.BlockSpec((1, CLS), lambda i: (0, 0)),
        ],
        out_specs=pl.BlockSpec((G, CLS), lambda i: (0, 0)),
        out_shape=jax.ShapeDtypeStruct((G, CLS), jnp.float32),
        scratch_shapes=[
            pltpu.VMEM((G, D), jnp.float32),
            pltpu.VMEM((G, D), jnp.float32),
        ],
    )(degp, accp, xsp, b, batch3, batch3, Wl1, bl1, Wl2, bl2)


def kernel(x, edge_index, batch, W1, b1, W2, b2, W3, b3, Wl1, bl1, Wl2, bl2):
    src = edge_index[0].astype(jnp.int32)
    dst = edge_index[1].astype(jnp.int32)
    srcp = jnp.where(src < NH, 2 * src, 2 * (src - NH) + 1)
    dstp = jnp.where(dst < NH, 2 * dst, 2 * (dst - NH) + 1)
    dstd = dstp.reshape(NW, NCHD, C)
    srcb = jnp.stack([srcp, srcp + N]).reshape(NC, NS, NCH, C)
    dstb = dstp.reshape(NS, NCH, C)
    batch3 = batch.astype(jnp.int32).reshape(N // TBH, 1, TBH)
    b1r, b2r, b3r = b1.reshape(1, D), b2.reshape(1, D), b3.reshape(1, D)
    bl1r, bl2r = bl1.reshape(1, D), bl2.reshape(1, CLS)

    degp = _deg_kernel(dstd).reshape(NC, NH, D)

    def segsum(xsp):
        return _segsum_kernel(xsp.reshape(NC * N, DH), srcb,
                              dstb).reshape(NC, NH, D)

    xs1 = _mm_scale(x, W1, degp)
    acc1 = segsum(xs1)
    xs2 = _combine(acc1, xs1, degp, b1r, W2)
    acc2 = segsum(xs2)
    xs3 = _combine(acc2, xs2, degp, b2r, W3)
    acc3 = segsum(xs3)
    return _final(acc3, xs3, degp, b3r, batch3, Wl1, bl1r, Wl2, bl2r)
